# baseline (speedup 1.0000x reference)
import functools

import jax
import jax.numpy as jnp
from jax import lax
from jax.experimental import pallas as pl
from jax.experimental.pallas import tpu as pltpu
import jax.experimental.pallas.tpu_sc as plsc

N, E, ND, ED, H, L = 10000, 320000, 128, 16, 128, 3
NC, NS = 2, 16
NW = NC * NS
GP = 2560
EP = GP * 128
TPW = GP // NW


def _worker_id():
  return lax.axis_index("s") * NC + lax.axis_index("c")


@functools.cache
def _sc_mesh():
  return plsc.VectorSubcoreMesh(
      core_axis_name="c", subcore_axis_name="s", num_cores=NC, num_subcores=NS)


def _gather_pair_body(table, idx_s, idx_d, out_s, out_d, idxv, rows, sem):
  w = _worker_id()
  g0 = w * TPW

  def group(t, carry):
    g = g0 + t
    pltpu.sync_copy(idx_s.at[g], idxv)
    pltpu.async_copy(table.at[idxv], rows, sem).wait()
    pltpu.sync_copy(rows, out_s.at[pl.ds(g * 128, 128)])
    pltpu.sync_copy(idx_d.at[g], idxv)
    pltpu.async_copy(table.at[idxv], rows, sem).wait()
    pltpu.sync_copy(rows, out_d.at[pl.ds(g * 128, 128)])
    return carry

  lax.fori_loop(0, TPW, group, 0)


@functools.cache
def _gather_pair_kernel():
  return pl.kernel(
      _gather_pair_body,
      out_type=(jax.ShapeDtypeStruct((EP, H), jnp.float32),
                jax.ShapeDtypeStruct((EP, H), jnp.float32)),
      mesh=_sc_mesh(),
      scratch_types=[
          pltpu.VMEM((128,), jnp.int32),
          pltpu.VMEM((128, H), jnp.float32),
          pltpu.SemaphoreType.DMA,
      ],
  )


def _scatter_add_body(m, idx_d, zeros, out, aggr, idxv, mv):
  c = lax.axis_index("c")
  s = lax.axis_index("s")
  w = s * NC + c
  g0 = w * TPW

  @pl.when(s < NS - 1)
  def _():
    pltpu.sync_copy(zeros.at[pl.ds(s * 640, 640)], aggr.at[pl.ds(s * 640, 640)])
  @pl.when(s == NS - 1)
  def _():
    pltpu.sync_copy(zeros.at[pl.ds(9600, 400)], aggr.at[pl.ds(9600, 400)])
  plsc.subcore_barrier()

  def group(t, carry):
    g = g0 + t
    pltpu.sync_copy(idx_d.at[g], idxv)
    pltpu.sync_copy(m.at[pl.ds(g * 128, 128)], mv)
    pltpu.sync_copy(mv, aggr.at[idxv], add=True)
    return carry

  lax.fori_loop(0, TPW, group, 0)
  plsc.subcore_barrier()

  @pl.when(s < NS - 1)
  def _():
    pltpu.sync_copy(aggr.at[pl.ds(s * 640, 640)], out.at[c, pl.ds(s * 640, 640)])
  @pl.when(s == NS - 1)
  def _():
    pltpu.sync_copy(aggr.at[pl.ds(9600, 400)], out.at[c, pl.ds(9600, 400)])


@functools.cache
def _scatter_add_kernel():
  return pl.kernel(
      _scatter_add_body,
      out_type=jax.ShapeDtypeStruct((NC, N, H), jnp.float32),
      mesh=_sc_mesh(),
      scratch_types=[
          pltpu.VMEM_SHARED((N, H), jnp.float32),
          pltpu.VMEM((128,), jnp.int32),
          pltpu.VMEM((128, H), jnp.float32),
      ],
  )


def _dot(a, b):
  return jnp.dot(a, b, preferred_element_type=jnp.float32)


def _ln_silu(t, g, b):
  mu = jnp.mean(t, axis=-1, keepdims=True)
  var = jnp.mean((t - mu) ** 2, axis=-1, keepdims=True)
  t = (t - mu) * lax.rsqrt(var + 1e-5) * g + b
  return t * jax.nn.sigmoid(t)


def _rep(shape):
  return pl.BlockSpec(shape, lambda i: (0, 0))


def _enc_n_body(x, w, b, out):
  out[...] = _dot(x[...], w[...]) + b[...]


def _enc_e_body(ea, w, b, out):
  out[...] = _dot(ea[...], w[...]) + b[...]


def _msggate_body(gd, gs, e, wma, wmb, wmc, bm1, gm1, bem1, wm2, bm2,
                  wga, wgb, wgc, bg1, gg1, beg1, wg2t, bg2, out):
  xi = gd[...]
  xj = gs[...]
  ev = e[...]
  t = _dot(xi, wma[...]) + _dot(xj, wmb[...]) + _dot(ev, wmc[...]) + bm1[...]
  t = _ln_silu(t, gm1[...], bem1[...])
  msg = _dot(t, wm2[...]) + bm2[...]
  u = _dot(xi, wga[...]) + _dot(xj, wgb[...]) + _dot(ev, wgc[...]) + bg1[...]
  u = _ln_silu(u, gg1[...], beg1[...])
  glog = jnp.sum(u * wg2t[...], axis=-1, keepdims=True) + bg2[...]
  m = jax.nn.sigmoid(glog) * msg
  row = pl.program_id(0) * BE + lax.broadcasted_iota(jnp.int32, m.shape, 0)
  out[...] = jnp.where(row < E, m, 0.0)


def _node_body(h, p0, p1, wa, wb, b1, g1, be1, w2, b2, out):
  hv = h[...]
  ag = p0[...] + p1[...]
  t = _dot(hv, wa[...]) + _dot(ag, wb[...]) + b1[...]
  t = _ln_silu(t, g1[...], be1[...])
  out[...] = _dot(t, w2[...]) + b2[...] + hv


def _edge_body(e, gs, gd, wa, wb, wc, b1, g1, be1, w2, b2, out):
  ev = e[...]
  t = _dot(ev, wa[...]) + _dot(gs[...], wb[...]) + _dot(gd[...], wc[...]) + b1[...]
  t = _ln_silu(t, g1[...], be1[...])
  out[...] = _dot(t, w2[...]) + b2[...] + ev


def _dec_n_body(h, w, b, out):
  out[...] = _dot(h[...], w[...]) + b[...]


def _dec_e_body(e, w, b, out):
  out[...] = _dot(e[...], w[...]) + b[...]


BE = 2048
BN = 2000


def _row_spec(b, d):
  return pl.BlockSpec((b, d), lambda i: (i, 0))


def _tc_call(body, n_rows, b, in_specs, out_dim):
  return pl.pallas_call(
      body,
      grid=(n_rows // b,),
      in_specs=in_specs,
      out_specs=_row_spec(b, out_dim),
      out_shape=jax.ShapeDtypeStruct((n_rows, out_dim), jnp.float32),
  )


def kernel(x, edge_index, edge_attr, enc_nW, enc_nb, enc_eW, enc_eb,
           msg_W1, msg_b1, msg_g1, msg_be1, msg_W2, msg_b2,
           gate_W1, gate_b1, gate_g1, gate_be1, gate_W2, gate_b2,
           node_W1, node_b1, node_g1, node_be1, node_W2, node_b2,
           edge_W1, edge_b1, edge_g1, edge_be1, edge_W2, edge_b2,
           dec_nW, dec_nb, dec_eW, dec_eb):
  ei = edge_index.astype(jnp.int32)
  pad = ((0, 0), (0, EP - E))
  src = jnp.pad(ei[0:1], pad).reshape(GP, 128)
  dst = jnp.pad(ei[1:2], pad).reshape(GP, 128)
  ea_p = jnp.pad(edge_attr, ((0, EP - E), (0, 0)))
  zeros = jnp.zeros((N, H), jnp.float32)
  r = lambda v: v.reshape(1, -1)

  enc_n = _tc_call(_enc_n_body, N, BN,
                   [_row_spec(BN, ND), _rep((ND, H)), _rep((1, H))], H)
  enc_e = _tc_call(_enc_e_body, EP, BE,
                   [_row_spec(BE, ED), _rep((ED, H)), _rep((1, H))], H)
  msggate = _tc_call(
      _msggate_body, EP, BE,
      [_row_spec(BE, H)] * 3
      + [_rep((H, H))] * 3 + [_rep((1, H))] * 3 + [_rep((H, H)), _rep((1, H))]
      + [_rep((H, H))] * 3 + [_rep((1, H))] * 3 + [_rep((1, H)), _rep((1, 1))],
      H)
  node = _tc_call(
      _node_body, N, BN,
      [_row_spec(BN, H)] * 3 + [_rep((H, H))] * 2 + [_rep((1, H))] * 3
      + [_rep((H, H)), _rep((1, H))], H)
  edge = _tc_call(
      _edge_body, EP, BE,
      [_row_spec(BE, H)] * 3 + [_rep((H, H))] * 3 + [_rep((1, H))] * 3
      + [_rep((H, H)), _rep((1, H))], H)
  dec_n = _tc_call(_dec_n_body, N, BN,
                   [_row_spec(BN, H), _rep((H, ND)), _rep((1, ND))], ND)
  dec_e = _tc_call(_dec_e_body, EP, BE,
                   [_row_spec(BE, H), _rep((H, ED)), _rep((1, ED))], ED)

  _gather_pair = _gather_pair_kernel()
  _scatter_add = _scatter_add_kernel()

  h = enc_n(x, enc_nW, r(enc_nb))
  e = enc_e(ea_p, enc_eW, r(enc_eb))
  gs, gd = _gather_pair(h, src, dst)

  for l in range(L):
    m = msggate(gd, gs, e,
                msg_W1[l, :H], msg_W1[l, H:2 * H], msg_W1[l, 2 * H:],
                r(msg_b1[l]), r(msg_g1[l]), r(msg_be1[l]),
                msg_W2[l], r(msg_b2[l]),
                gate_W1[l, :H], gate_W1[l, H:2 * H], gate_W1[l, 2 * H:],
                r(gate_b1[l]), r(gate_g1[l]), r(gate_be1[l]),
                gate_W2[l].reshape(1, H), gate_b2[l].reshape(1, 1))
    parts = _scatter_add(m, dst, zeros)
    h = node(h, parts[0], parts[1],
             node_W1[l, :H], node_W1[l, H:],
             r(node_b1[l]), r(node_g1[l]), r(node_be1[l]),
             node_W2[l], r(node_b2[l]))
    gs, gd = _gather_pair(h, src, dst)
    e = edge(e, gs, gd,
             edge_W1[l, :H], edge_W1[l, H:2 * H], edge_W1[l, 2 * H:],
             r(edge_b1[l]), r(edge_g1[l]), r(edge_be1[l]),
             edge_W2[l], r(edge_b2[l]))

  x_out = dec_n(h, dec_nW, r(dec_nb))
  e_out = dec_e(e, dec_eW, r(dec_eb))
  return (x_out, e_out[:E])


# speedup vs baseline: 1.6247x; 1.6247x over previous
import functools

import jax
import jax.numpy as jnp
from jax import lax
from jax.experimental import pallas as pl
from jax.experimental.pallas import tpu as pltpu
import jax.experimental.pallas.tpu_sc as plsc

N, E, ND, ED, H, L = 10000, 320000, 128, 16, 128, 3
NC, NS = 2, 16
NW = NC * NS
G = E // 128


def _worker_id():
  return lax.axis_index("s") * NC + lax.axis_index("c")


@functools.cache
def _sc_mesh():
  return plsc.VectorSubcoreMesh(
      core_axis_name="c", subcore_axis_name="s", num_cores=NC, num_subcores=NS)


def _gather_pair_body(table, idx_s, idx_d, out_s, out_d, idxv, rows, sem):
  w = _worker_id()
  g0 = (w * G) // NW
  g1 = ((w + 1) * G) // NW

  def group(g, carry):
    pltpu.sync_copy(idx_s.at[g], idxv)
    pltpu.async_copy(table.at[idxv], rows, sem).wait()
    pltpu.sync_copy(rows, out_s.at[pl.ds(g * 128, 128)])
    pltpu.sync_copy(idx_d.at[g], idxv)
    pltpu.async_copy(table.at[idxv], rows, sem).wait()
    pltpu.sync_copy(rows, out_d.at[pl.ds(g * 128, 128)])
    return carry

  lax.fori_loop(g0, g1, group, 0)


@functools.cache
def _gather_pair_kernel():
  return pl.kernel(
      _gather_pair_body,
      out_type=(jax.ShapeDtypeStruct((E, H), jnp.float32),
                jax.ShapeDtypeStruct((E, H), jnp.float32)),
      mesh=_sc_mesh(),
      scratch_types=[
          pltpu.VMEM((128,), jnp.int32),
          pltpu.VMEM((128, H), jnp.float32),
          pltpu.SemaphoreType.DMA,
      ],
  )


def _scatter_add_body(m, idx_d, zeros, out, aggr, idxv, mv):
  c = lax.axis_index("c")
  s = lax.axis_index("s")
  w = s * NC + c

  @pl.when(s < NS - 1)
  def _():
    pltpu.sync_copy(zeros.at[pl.ds(s * 640, 640)], aggr.at[pl.ds(s * 640, 640)])
  @pl.when(s == NS - 1)
  def _():
    pltpu.sync_copy(zeros.at[pl.ds(9600, 400)], aggr.at[pl.ds(9600, 400)])
  plsc.subcore_barrier()
  g0 = (w * G) // NW
  g1 = ((w + 1) * G) // NW

  def group(g, carry):
    pltpu.sync_copy(idx_d.at[g], idxv)
    pltpu.sync_copy(m.at[pl.ds(g * 128, 128)], mv)
    pltpu.sync_copy(mv, aggr.at[idxv], add=True)
    return carry

  lax.fori_loop(g0, g1, group, 0)
  plsc.subcore_barrier()

  @pl.when(s < NS - 1)
  def _():
    pltpu.sync_copy(aggr.at[pl.ds(s * 640, 640)], out.at[c, pl.ds(s * 640, 640)])
  @pl.when(s == NS - 1)
  def _():
    pltpu.sync_copy(aggr.at[pl.ds(9600, 400)], out.at[c, pl.ds(9600, 400)])


@functools.cache
def _scatter_add_kernel():
  return pl.kernel(
      _scatter_add_body,
      out_type=jax.ShapeDtypeStruct((NC, N, H), jnp.float32),
      mesh=_sc_mesh(),
      scratch_types=[
          pltpu.VMEM_SHARED((N, H), jnp.float32),
          pltpu.VMEM((128,), jnp.int32),
          pltpu.VMEM((128, H), jnp.float32),
      ],
  )


def _dot(a, b):
  return jnp.dot(a, b, preferred_element_type=jnp.float32)


def _ln_silu(t, g, b):
  mu = jnp.mean(t, axis=-1, keepdims=True)
  var = jnp.mean((t - mu) ** 2, axis=-1, keepdims=True)
  t = (t - mu) * lax.rsqrt(var + 1e-5) * g + b
  return t * jax.nn.sigmoid(t)


def _rep(shape):
  return pl.BlockSpec(shape, lambda i: (0, 0))


def _enc_n_body(x, w, b, out):
  out[...] = _dot(x[...], w[...]) + b[...]


def _enc_e_body(ea, w, b, out):
  out[...] = _dot(ea[...], w[...]) + b[...]


def _msggate_body(gd, gs, e, wma, wmb, wmc, bm1, gm1, bem1, wm2, bm2,
                  wga, wgb, wgc, bg1, gg1, beg1, wg2t, bg2, out):
  xi = gd[...]
  xj = gs[...]
  ev = e[...]
  t = _dot(xi, wma[...]) + _dot(xj, wmb[...]) + _dot(ev, wmc[...]) + bm1[...]
  t = _ln_silu(t, gm1[...], bem1[...])
  msg = _dot(t, wm2[...]) + bm2[...]
  u = _dot(xi, wga[...]) + _dot(xj, wgb[...]) + _dot(ev, wgc[...]) + bg1[...]
  u = _ln_silu(u, gg1[...], beg1[...])
  glog = jnp.sum(u * wg2t[...], axis=-1, keepdims=True) + bg2[...]
  out[...] = jax.nn.sigmoid(glog) * msg


def _node_body(h, p0, p1, wa, wb, b1, g1, be1, w2, b2, out):
  hv = h[...]
  ag = p0[...] + p1[...]
  t = _dot(hv, wa[...]) + _dot(ag, wb[...]) + b1[...]
  t = _ln_silu(t, g1[...], be1[...])
  out[...] = _dot(t, w2[...]) + b2[...] + hv


def _edge_body(e, gs, gd, wa, wb, wc, b1, g1, be1, w2, b2, out):
  ev = e[...]
  t = _dot(ev, wa[...]) + _dot(gs[...], wb[...]) + _dot(gd[...], wc[...]) + b1[...]
  t = _ln_silu(t, g1[...], be1[...])
  out[...] = _dot(t, w2[...]) + b2[...] + ev


def _dec_n_body(h, w, b, out):
  out[...] = _dot(h[...], w[...]) + b[...]


def _dec_e_body(e, w, b, out):
  out[...] = _dot(e[...], w[...]) + b[...]


BE = 2000
BN = 2000


def _row_spec(b, d):
  return pl.BlockSpec((b, d), lambda i: (i, 0))


def _tc_call(body, n_rows, b, in_specs, out_dim):
  return pl.pallas_call(
      body,
      grid=(n_rows // b,),
      in_specs=in_specs,
      out_specs=_row_spec(b, out_dim),
      out_shape=jax.ShapeDtypeStruct((n_rows, out_dim), jnp.float32),
  )


def kernel(x, edge_index, edge_attr, enc_nW, enc_nb, enc_eW, enc_eb,
           msg_W1, msg_b1, msg_g1, msg_be1, msg_W2, msg_b2,
           gate_W1, gate_b1, gate_g1, gate_be1, gate_W2, gate_b2,
           node_W1, node_b1, node_g1, node_be1, node_W2, node_b2,
           edge_W1, edge_b1, edge_g1, edge_be1, edge_W2, edge_b2,
           dec_nW, dec_nb, dec_eW, dec_eb):
  src = edge_index[0].astype(jnp.int32).reshape(G, 128)
  dst = edge_index[1].astype(jnp.int32).reshape(G, 128)
  zeros = jnp.zeros((N, H), jnp.float32)
  r = lambda v: v.reshape(1, -1)

  enc_n = _tc_call(_enc_n_body, N, BN,
                   [_row_spec(BN, ND), _rep((ND, H)), _rep((1, H))], H)
  enc_e = _tc_call(_enc_e_body, E, BE,
                   [_row_spec(BE, ED), _rep((ED, H)), _rep((1, H))], H)
  msggate = _tc_call(
      _msggate_body, E, BE,
      [_row_spec(BE, H)] * 3
      + [_rep((H, H))] * 3 + [_rep((1, H))] * 3 + [_rep((H, H)), _rep((1, H))]
      + [_rep((H, H))] * 3 + [_rep((1, H))] * 3 + [_rep((1, H)), _rep((1, 1))],
      H)
  node = _tc_call(
      _node_body, N, BN,
      [_row_spec(BN, H)] * 3 + [_rep((H, H))] * 2 + [_rep((1, H))] * 3
      + [_rep((H, H)), _rep((1, H))], H)
  edge = _tc_call(
      _edge_body, E, BE,
      [_row_spec(BE, H)] * 3 + [_rep((H, H))] * 3 + [_rep((1, H))] * 3
      + [_rep((H, H)), _rep((1, H))], H)
  dec_n = _tc_call(_dec_n_body, N, BN,
                   [_row_spec(BN, H), _rep((H, ND)), _rep((1, ND))], ND)
  dec_e = _tc_call(_dec_e_body, E, BE,
                   [_row_spec(BE, H), _rep((H, ED)), _rep((1, ED))], ED)

  _gather_pair = _gather_pair_kernel()
  _scatter_add = _scatter_add_kernel()

  h = enc_n(x, enc_nW, r(enc_nb))
  e = enc_e(edge_attr, enc_eW, r(enc_eb))
  gs, gd = _gather_pair(h, src, dst)

  for l in range(L):
    m = msggate(gd, gs, e,
                msg_W1[l, :H], msg_W1[l, H:2 * H], msg_W1[l, 2 * H:],
                r(msg_b1[l]), r(msg_g1[l]), r(msg_be1[l]),
                msg_W2[l], r(msg_b2[l]),
                gate_W1[l, :H], gate_W1[l, H:2 * H], gate_W1[l, 2 * H:],
                r(gate_b1[l]), r(gate_g1[l]), r(gate_be1[l]),
                gate_W2[l].reshape(1, H), gate_b2[l].reshape(1, 1))
    parts = _scatter_add(m, dst, zeros)
    h = node(h, parts[0], parts[1],
             node_W1[l, :H], node_W1[l, H:],
             r(node_b1[l]), r(node_g1[l]), r(node_be1[l]),
             node_W2[l], r(node_b2[l]))
    gs, gd = _gather_pair(h, src, dst)
    e = edge(e, gs, gd,
             edge_W1[l, :H], edge_W1[l, H:2 * H], edge_W1[l, 2 * H:],
             r(edge_b1[l]), r(edge_g1[l]), r(edge_be1[l]),
             edge_W2[l], r(edge_b2[l]))

  x_out = dec_n(h, dec_nW, r(dec_nb))
  e_out = dec_e(e, dec_eW, r(dec_eb))
  return (x_out, e_out)


# speedup vs baseline: 1.6547x; 1.0185x over previous
import functools

import jax
import jax.numpy as jnp
from jax import lax
from jax.experimental import pallas as pl
from jax.experimental.pallas import tpu as pltpu
import jax.experimental.pallas.tpu_sc as plsc

N, E, ND, ED, H, L = 10000, 320000, 128, 16, 128, 3
NC, NS = 2, 16
NW = NC * NS
G = E // 128


def _worker_id():
  return lax.axis_index("s") * NC + lax.axis_index("c")


@functools.cache
def _sc_mesh():
  return plsc.VectorSubcoreMesh(
      core_axis_name="c", subcore_axis_name="s", num_cores=NC, num_subcores=NS)


def _gather_pair_body(table, idx_s, idx_d, out_s, out_d, idxv, rows, sem):
  w = _worker_id()
  g0 = (w * G) // NW
  g1 = ((w + 1) * G) // NW

  def group(g, carry):
    pltpu.sync_copy(idx_s.at[g], idxv)
    pltpu.async_copy(table.at[idxv], rows, sem).wait()
    pltpu.sync_copy(rows, out_s.at[pl.ds(g * 128, 128)])
    pltpu.sync_copy(idx_d.at[g], idxv)
    pltpu.async_copy(table.at[idxv], rows, sem).wait()
    pltpu.sync_copy(rows, out_d.at[pl.ds(g * 128, 128)])
    return carry

  lax.fori_loop(g0, g1, group, 0)


@functools.cache
def _gather_pair_kernel():
  return pl.kernel(
      _gather_pair_body,
      out_type=(jax.ShapeDtypeStruct((E, H), jnp.float32),
                jax.ShapeDtypeStruct((E, H), jnp.float32)),
      mesh=_sc_mesh(),
      scratch_types=[
          pltpu.VMEM((128,), jnp.int32),
          pltpu.VMEM((128, H), jnp.float32),
          pltpu.SemaphoreType.DMA,
      ],
  )


def _scatter_add_body(m, idx_d, zeros, out, aggr, idxv, mv):
  c = lax.axis_index("c")
  s = lax.axis_index("s")
  w = s * NC + c

  @pl.when(s < NS - 1)
  def _():
    pltpu.sync_copy(zeros.at[pl.ds(s * 640, 640)], aggr.at[pl.ds(s * 640, 640)])
  @pl.when(s == NS - 1)
  def _():
    pltpu.sync_copy(zeros.at[pl.ds(9600, 400)], aggr.at[pl.ds(9600, 400)])
  plsc.subcore_barrier()
  g0 = (w * G) // NW
  g1 = ((w + 1) * G) // NW

  def group(g, carry):
    pltpu.sync_copy(idx_d.at[g], idxv)
    pltpu.sync_copy(m.at[pl.ds(g * 128, 128)], mv)
    pltpu.sync_copy(mv, aggr.at[idxv], add=True)
    return carry

  lax.fori_loop(g0, g1, group, 0)
  plsc.subcore_barrier()

  @pl.when(s < NS - 1)
  def _():
    pltpu.sync_copy(aggr.at[pl.ds(s * 640, 640)], out.at[c, pl.ds(s * 640, 640)])
  @pl.when(s == NS - 1)
  def _():
    pltpu.sync_copy(aggr.at[pl.ds(9600, 400)], out.at[c, pl.ds(9600, 400)])


@functools.cache
def _scatter_add_kernel():
  return pl.kernel(
      _scatter_add_body,
      out_type=jax.ShapeDtypeStruct((NC, N, H), jnp.float32),
      mesh=_sc_mesh(),
      scratch_types=[
          pltpu.VMEM_SHARED((N, H), jnp.float32),
          pltpu.VMEM((128,), jnp.int32),
          pltpu.VMEM((128, H), jnp.float32),
      ],
  )


def _dot(a, b):
  return jnp.dot(a, b, preferred_element_type=jnp.float32)


def _ln_silu(t, g, b):
  mu = jnp.mean(t, axis=-1, keepdims=True)
  var = jnp.mean((t - mu) ** 2, axis=-1, keepdims=True)
  t = (t - mu) * lax.rsqrt(var + 1e-5) * g + b
  return t * jax.nn.sigmoid(t)


def _rep(shape):
  return pl.BlockSpec(shape, lambda i: (0, 0))


def _enc_n_body(x, w, b, out):
  out[...] = _dot(x[...], w[...]) + b[...]


def _enc_e_body(ea, w, b, out):
  out[...] = _dot(ea[...], w[...]) + b[...]


def _msggate_body(gd, gs, e, wma, wmb, wmc, bm1, gm1, bem1, wm2, bm2,
                  wga, wgb, wgc, bg1, gg1, beg1, wg2t, bg2, out):
  xi = gd[...]
  xj = gs[...]
  ev = e[...]
  t = _dot(xi, wma[...]) + _dot(xj, wmb[...]) + _dot(ev, wmc[...]) + bm1[...]
  t = _ln_silu(t, gm1[...], bem1[...])
  msg = _dot(t, wm2[...]) + bm2[...]
  u = _dot(xi, wga[...]) + _dot(xj, wgb[...]) + _dot(ev, wgc[...]) + bg1[...]
  u = _ln_silu(u, gg1[...], beg1[...])
  glog = jnp.sum(u * wg2t[...], axis=-1, keepdims=True) + bg2[...]
  out[...] = jax.nn.sigmoid(glog) * msg


def _node_body(h, p0, p1, wa, wb, b1, g1, be1, w2, b2, out):
  hv = h[...]
  ag = p0[...] + p1[...]
  t = _dot(hv, wa[...]) + _dot(ag, wb[...]) + b1[...]
  t = _ln_silu(t, g1[...], be1[...])
  out[...] = _dot(t, w2[...]) + b2[...] + hv


def _edge_body(e, gs, gd, wa, wb, wc, b1, g1, be1, w2, b2, out):
  ev = e[...]
  t = _dot(ev, wa[...]) + _dot(gs[...], wb[...]) + _dot(gd[...], wc[...]) + b1[...]
  t = _ln_silu(t, g1[...], be1[...])
  out[...] = _dot(t, w2[...]) + b2[...] + ev


def _dec_n_body(h, w, b, out):
  out[...] = _dot(h[...], w[...]) + b[...]


def _dec_e_body(e, w, b, out):
  out[...] = _dot(e[...], w[...]) + b[...]


BE = 4000
BN = 2000


def _row_spec(b, d):
  return pl.BlockSpec((b, d), lambda i: (i, 0))


def _tc_call(body, n_rows, b, in_specs, out_dim):
  return pl.pallas_call(
      body,
      grid=(n_rows // b,),
      in_specs=in_specs,
      out_specs=_row_spec(b, out_dim),
      out_shape=jax.ShapeDtypeStruct((n_rows, out_dim), jnp.float32),
  )


def kernel(x, edge_index, edge_attr, enc_nW, enc_nb, enc_eW, enc_eb,
           msg_W1, msg_b1, msg_g1, msg_be1, msg_W2, msg_b2,
           gate_W1, gate_b1, gate_g1, gate_be1, gate_W2, gate_b2,
           node_W1, node_b1, node_g1, node_be1, node_W2, node_b2,
           edge_W1, edge_b1, edge_g1, edge_be1, edge_W2, edge_b2,
           dec_nW, dec_nb, dec_eW, dec_eb):
  src = edge_index[0].astype(jnp.int32).reshape(G, 128)
  dst = edge_index[1].astype(jnp.int32).reshape(G, 128)
  zeros = jnp.zeros((N, H), jnp.float32)
  r = lambda v: v.reshape(1, -1)

  enc_n = _tc_call(_enc_n_body, N, BN,
                   [_row_spec(BN, ND), _rep((ND, H)), _rep((1, H))], H)
  enc_e = _tc_call(_enc_e_body, E, BE,
                   [_row_spec(BE, ED), _rep((ED, H)), _rep((1, H))], H)
  msggate = _tc_call(
      _msggate_body, E, BE,
      [_row_spec(BE, H)] * 3
      + [_rep((H, H))] * 3 + [_rep((1, H))] * 3 + [_rep((H, H)), _rep((1, H))]
      + [_rep((H, H))] * 3 + [_rep((1, H))] * 3 + [_rep((1, H)), _rep((1, 1))],
      H)
  node = _tc_call(
      _node_body, N, BN,
      [_row_spec(BN, H)] * 3 + [_rep((H, H))] * 2 + [_rep((1, H))] * 3
      + [_rep((H, H)), _rep((1, H))], H)
  edge = _tc_call(
      _edge_body, E, BE,
      [_row_spec(BE, H)] * 3 + [_rep((H, H))] * 3 + [_rep((1, H))] * 3
      + [_rep((H, H)), _rep((1, H))], H)
  dec_n = _tc_call(_dec_n_body, N, BN,
                   [_row_spec(BN, H), _rep((H, ND)), _rep((1, ND))], ND)
  dec_e = _tc_call(_dec_e_body, E, BE,
                   [_row_spec(BE, H), _rep((H, ED)), _rep((1, ED))], ED)

  _gather_pair = _gather_pair_kernel()
  _scatter_add = _scatter_add_kernel()

  h = enc_n(x, enc_nW, r(enc_nb))
  e = enc_e(edge_attr, enc_eW, r(enc_eb))
  gs, gd = _gather_pair(h, src, dst)

  for l in range(L):
    m = msggate(gd, gs, e,
                msg_W1[l, :H], msg_W1[l, H:2 * H], msg_W1[l, 2 * H:],
                r(msg_b1[l]), r(msg_g1[l]), r(msg_be1[l]),
                msg_W2[l], r(msg_b2[l]),
                gate_W1[l, :H], gate_W1[l, H:2 * H], gate_W1[l, 2 * H:],
                r(gate_b1[l]), r(gate_g1[l]), r(gate_be1[l]),
                gate_W2[l].reshape(1, H), gate_b2[l].reshape(1, 1))
    parts = _scatter_add(m, dst, zeros)
    h = node(h, parts[0], parts[1],
             node_W1[l, :H], node_W1[l, H:],
             r(node_b1[l]), r(node_g1[l]), r(node_be1[l]),
             node_W2[l], r(node_b2[l]))
    gs, gd = _gather_pair(h, src, dst)
    e = edge(e, gs, gd,
             edge_W1[l, :H], edge_W1[l, H:2 * H], edge_W1[l, 2 * H:],
             r(edge_b1[l]), r(edge_g1[l]), r(edge_be1[l]),
             edge_W2[l], r(edge_b2[l]))

  x_out = dec_n(h, dec_nW, r(dec_nb))
  e_out = dec_e(e, dec_eW, r(dec_eb))
  return (x_out, e_out)


# speedup vs baseline: 1.6567x; 1.0012x over previous
import functools

import jax
import jax.numpy as jnp
from jax import lax
from jax.experimental import pallas as pl
from jax.experimental.pallas import tpu as pltpu
import jax.experimental.pallas.tpu_sc as plsc

N, E, ND, ED, H, L = 10000, 320000, 128, 16, 128, 3
NC, NS = 2, 16
NW = NC * NS
G = E // 128


def _worker_id():
  return lax.axis_index("s") * NC + lax.axis_index("c")


@functools.cache
def _sc_mesh():
  return plsc.VectorSubcoreMesh(
      core_axis_name="c", subcore_axis_name="s", num_cores=NC, num_subcores=NS)


def _gather_pair_body(table, idx_s, idx_d, out_s, out_d, idxv, rows, sem):
  w = _worker_id()
  g0 = (w * G) // NW
  g1 = ((w + 1) * G) // NW

  def group(g, carry):
    pltpu.sync_copy(idx_s.at[g], idxv)
    pltpu.async_copy(table.at[idxv], rows, sem).wait()
    pltpu.sync_copy(rows, out_s.at[pl.ds(g * 128, 128)])
    pltpu.sync_copy(idx_d.at[g], idxv)
    pltpu.async_copy(table.at[idxv], rows, sem).wait()
    pltpu.sync_copy(rows, out_d.at[pl.ds(g * 128, 128)])
    return carry

  lax.fori_loop(g0, g1, group, 0)


@functools.cache
def _gather_pair_kernel():
  return pl.kernel(
      _gather_pair_body,
      out_type=(jax.ShapeDtypeStruct((E, H), jnp.float32),
                jax.ShapeDtypeStruct((E, H), jnp.float32)),
      mesh=_sc_mesh(),
      scratch_types=[
          pltpu.VMEM((128,), jnp.int32),
          pltpu.VMEM((128, H), jnp.float32),
          pltpu.SemaphoreType.DMA,
      ],
  )


def _scatter_add_body(m, idx_d, zeros, out, aggr, idxv, mv):
  c = lax.axis_index("c")
  s = lax.axis_index("s")
  w = s * NC + c

  @pl.when(s < NS - 1)
  def _():
    pltpu.sync_copy(zeros.at[pl.ds(s * 640, 640)], aggr.at[pl.ds(s * 640, 640)])
  @pl.when(s == NS - 1)
  def _():
    pltpu.sync_copy(zeros.at[pl.ds(9600, 400)], aggr.at[pl.ds(9600, 400)])
  plsc.subcore_barrier()
  g0 = (w * G) // NW
  g1 = ((w + 1) * G) // NW

  def group(g, carry):
    pltpu.sync_copy(idx_d.at[g], idxv)
    pltpu.sync_copy(m.at[pl.ds(g * 128, 128)], mv)
    pltpu.sync_copy(mv, aggr.at[idxv], add=True)
    return carry

  lax.fori_loop(g0, g1, group, 0)
  plsc.subcore_barrier()

  @pl.when(s < NS - 1)
  def _():
    pltpu.sync_copy(aggr.at[pl.ds(s * 640, 640)], out.at[c, pl.ds(s * 640, 640)])
  @pl.when(s == NS - 1)
  def _():
    pltpu.sync_copy(aggr.at[pl.ds(9600, 400)], out.at[c, pl.ds(9600, 400)])


@functools.cache
def _scatter_add_kernel():
  return pl.kernel(
      _scatter_add_body,
      out_type=jax.ShapeDtypeStruct((NC, N, H), jnp.float32),
      mesh=_sc_mesh(),
      scratch_types=[
          pltpu.VMEM_SHARED((N, H), jnp.float32),
          pltpu.VMEM((128,), jnp.int32),
          pltpu.VMEM((128, H), jnp.float32),
      ],
  )


def _dot(a, b):
  return jnp.dot(a, b, preferred_element_type=jnp.float32)


def _ln_silu(t, g, b):
  mu = jnp.mean(t, axis=-1, keepdims=True)
  var = jnp.mean((t - mu) ** 2, axis=-1, keepdims=True)
  t = (t - mu) * lax.rsqrt(var + 1e-5) * g + b
  return t * jax.nn.sigmoid(t)


def _rep(shape):
  return pl.BlockSpec(shape, lambda i: (0, 0))


def _enc_n_body(x, w, b, out):
  out[...] = _dot(x[...], w[...]) + b[...]


def _enc_e_body(ea, w, b, out):
  out[...] = _dot(ea[...], w[...]) + b[...]


def _msggate_body(gd, gs, e, wma, wmb, wmc, bm1, gm1, bem1, wm2, bm2,
                  wga, wgb, wgc, bg1, gg1, beg1, wg2t, bg2, out):
  xi = gd[...]
  xj = gs[...]
  ev = e[...]
  t = _dot(xi, wma[...]) + _dot(xj, wmb[...]) + _dot(ev, wmc[...]) + bm1[...]
  t = _ln_silu(t, gm1[...], bem1[...])
  msg = _dot(t, wm2[...]) + bm2[...]
  u = _dot(xi, wga[...]) + _dot(xj, wgb[...]) + _dot(ev, wgc[...]) + bg1[...]
  u = _ln_silu(u, gg1[...], beg1[...])
  glog = jnp.sum(u * wg2t[...], axis=-1, keepdims=True) + bg2[...]
  out[...] = jax.nn.sigmoid(glog) * msg


def _node_body(h, p0, p1, wa, wb, b1, g1, be1, w2, b2, out):
  hv = h[...]
  ag = p0[...] + p1[...]
  t = _dot(hv, wa[...]) + _dot(ag, wb[...]) + b1[...]
  t = _ln_silu(t, g1[...], be1[...])
  out[...] = _dot(t, w2[...]) + b2[...] + hv


def _edge_body(e, gs, gd, wa, wb, wc, b1, g1, be1, w2, b2, out):
  ev = e[...]
  t = _dot(ev, wa[...]) + _dot(gs[...], wb[...]) + _dot(gd[...], wc[...]) + b1[...]
  t = _ln_silu(t, g1[...], be1[...])
  out[...] = _dot(t, w2[...]) + b2[...] + ev


def _dec_n_body(h, w, b, out):
  out[...] = _dot(h[...], w[...]) + b[...]


def _dec_e_body(e, w, b, out):
  out[...] = _dot(e[...], w[...]) + b[...]


BE = 8000
BN = 2000


def _row_spec(b, d):
  return pl.BlockSpec((b, d), lambda i: (i, 0))


def _tc_call(body, n_rows, b, in_specs, out_dim):
  return pl.pallas_call(
      body,
      grid=(n_rows // b,),
      in_specs=in_specs,
      out_specs=_row_spec(b, out_dim),
      out_shape=jax.ShapeDtypeStruct((n_rows, out_dim), jnp.float32),
  )


def kernel(x, edge_index, edge_attr, enc_nW, enc_nb, enc_eW, enc_eb,
           msg_W1, msg_b1, msg_g1, msg_be1, msg_W2, msg_b2,
           gate_W1, gate_b1, gate_g1, gate_be1, gate_W2, gate_b2,
           node_W1, node_b1, node_g1, node_be1, node_W2, node_b2,
           edge_W1, edge_b1, edge_g1, edge_be1, edge_W2, edge_b2,
           dec_nW, dec_nb, dec_eW, dec_eb):
  src = edge_index[0].astype(jnp.int32).reshape(G, 128)
  dst = edge_index[1].astype(jnp.int32).reshape(G, 128)
  zeros = jnp.zeros((N, H), jnp.float32)
  r = lambda v: v.reshape(1, -1)

  enc_n = _tc_call(_enc_n_body, N, BN,
                   [_row_spec(BN, ND), _rep((ND, H)), _rep((1, H))], H)
  enc_e = _tc_call(_enc_e_body, E, BE,
                   [_row_spec(BE, ED), _rep((ED, H)), _rep((1, H))], H)
  msggate = _tc_call(
      _msggate_body, E, BE,
      [_row_spec(BE, H)] * 3
      + [_rep((H, H))] * 3 + [_rep((1, H))] * 3 + [_rep((H, H)), _rep((1, H))]
      + [_rep((H, H))] * 3 + [_rep((1, H))] * 3 + [_rep((1, H)), _rep((1, 1))],
      H)
  node = _tc_call(
      _node_body, N, BN,
      [_row_spec(BN, H)] * 3 + [_rep((H, H))] * 2 + [_rep((1, H))] * 3
      + [_rep((H, H)), _rep((1, H))], H)
  edge = _tc_call(
      _edge_body, E, BE,
      [_row_spec(BE, H)] * 3 + [_rep((H, H))] * 3 + [_rep((1, H))] * 3
      + [_rep((H, H)), _rep((1, H))], H)
  dec_n = _tc_call(_dec_n_body, N, BN,
                   [_row_spec(BN, H), _rep((H, ND)), _rep((1, ND))], ND)
  dec_e = _tc_call(_dec_e_body, E, BE,
                   [_row_spec(BE, H), _rep((H, ED)), _rep((1, ED))], ED)

  _gather_pair = _gather_pair_kernel()
  _scatter_add = _scatter_add_kernel()

  h = enc_n(x, enc_nW, r(enc_nb))
  e = enc_e(edge_attr, enc_eW, r(enc_eb))
  gs, gd = _gather_pair(h, src, dst)

  for l in range(L):
    m = msggate(gd, gs, e,
                msg_W1[l, :H], msg_W1[l, H:2 * H], msg_W1[l, 2 * H:],
                r(msg_b1[l]), r(msg_g1[l]), r(msg_be1[l]),
                msg_W2[l], r(msg_b2[l]),
                gate_W1[l, :H], gate_W1[l, H:2 * H], gate_W1[l, 2 * H:],
                r(gate_b1[l]), r(gate_g1[l]), r(gate_be1[l]),
                gate_W2[l].reshape(1, H), gate_b2[l].reshape(1, 1))
    parts = _scatter_add(m, dst, zeros)
    h = node(h, parts[0], parts[1],
             node_W1[l, :H], node_W1[l, H:],
             r(node_b1[l]), r(node_g1[l]), r(node_be1[l]),
             node_W2[l], r(node_b2[l]))
    gs, gd = _gather_pair(h, src, dst)
    e = edge(e, gs, gd,
             edge_W1[l, :H], edge_W1[l, H:2 * H], edge_W1[l, 2 * H:],
             r(edge_b1[l]), r(edge_g1[l]), r(edge_be1[l]),
             edge_W2[l], r(edge_b2[l]))

  x_out = dec_n(h, dec_nW, r(dec_nb))
  e_out = dec_e(e, dec_eW, r(dec_eb))
  return (x_out, e_out)
